# Initial kernel scaffold; baseline (speedup 1.0000x reference)
import jax
import jax.numpy as jnp
from jax.experimental import pallas as pl


def kernel(x, edge_index, W1l, b1, W1r, W2l, b2, W2r):
    raise NotImplementedError("write your pallas kernel here")


# speedup vs baseline: 4.4421x; 4.4421x over previous
import jax
import jax.numpy as jnp
from jax import lax
from jax.experimental import pallas as pl
from jax.experimental.pallas import tpu as pltpu
from jax.experimental.pallas import tpu_sc as plsc

N = 10000
D = 128
E = 320000

NC, NS = 2, 16
NW = NC * NS
CHUNK = 128
N_CHUNKS = -(-E // (NW * CHUNK))
E_PAD = NW * CHUNK * N_CHUNKS
PE = E_PAD // NW
N_PAD = 10112
RPT = N_PAD // NS

_DN = (((1,), (1,)), ((), ()))


def _make_deg_count():
    mesh = plsc.VectorSubcoreMesh(core_axis_name="c", subcore_axis_name="s",
                                  num_cores=NC, num_subcores=NS)

    def body(dst_hbm, dego_hbm, dst_v, deg_v):
        c = lax.axis_index("c")
        s = lax.axis_index("s")
        w = c * NS + s
        pltpu.sync_copy(dst_hbm.at[pl.ds(w * PE, PE)], dst_v)

        def zero_deg(i, carry):
            deg_v[pl.ds(i * 16, 16)] = jnp.zeros((16,), jnp.float32)
            return carry

        lax.fori_loop(0, N_PAD // 16, zero_deg, 0)

        def count(i, carry):
            idx16 = dst_v[pl.ds(i * 16, 16)]
            cnt, last = plsc.scan_count(idx16)
            plsc.addupdate_scatter(deg_v, [idx16],
                                   cnt.astype(jnp.float32), mask=last)
            return carry

        lax.fori_loop(0, PE // 16, count, 0)
        pltpu.sync_copy(deg_v, dego_hbm.at[pl.ds(w * N_PAD, N_PAD)])

    return pl.kernel(
        body,
        out_type=(jax.ShapeDtypeStruct((NW * N_PAD,), jnp.float32),),
        mesh=mesh,
        compiler_params=pltpu.CompilerParams(needs_layout_passes=False),
        scratch_types=(
            pltpu.VMEM((PE,), jnp.int32),
            pltpu.VMEM((N_PAD,), jnp.float32),
        ),
    )


def _make_seg_sum():
    mesh = plsc.VectorSubcoreMesh(core_axis_name="c", subcore_axis_name="s",
                                  num_cores=NC, num_subcores=NS)

    def body(*args):
        (xt_hbm, src_hbm, dst_hbm, out_hbm,
         src_v, dst_v, rows_v, acc_sh, sem) = args
        c = lax.axis_index("c")
        s = lax.axis_index("s")
        w = c * NS + s
        rbase = s * RPT
        n_full = RPT // CHUNK
        tail = RPT - n_full * CHUNK

        def zero_rows(i, carry):
            for j in range(D // 16):
                rows_v[i, pl.ds(j * 16, 16)] = jnp.zeros((16,), jnp.float32)
            return carry

        lax.fori_loop(0, CHUNK, zero_rows, 0)

        def zcopy(i, carry):
            pltpu.sync_copy(rows_v, acc_sh.at[pl.ds(rbase + i * CHUNK, CHUNK)])
            return carry

        lax.fori_loop(0, n_full, zcopy, 0)
        pltpu.sync_copy(rows_v.at[pl.ds(0, tail)],
                        acc_sh.at[pl.ds(rbase + n_full * CHUNK, tail)])
        plsc.subcore_barrier()

        ebase = w * PE

        def step(i, carry):
            b = ebase + i * CHUNK
            pltpu.sync_copy(src_hbm.at[pl.ds(b, CHUNK)], src_v)
            pltpu.sync_copy(dst_hbm.at[pl.ds(b, CHUNK)], dst_v)
            pltpu.async_copy(xt_hbm.at[src_v], rows_v, sem).wait()
            pltpu.sync_copy(rows_v, acc_sh.at[dst_v], add=True)
            return carry

        lax.fori_loop(0, N_CHUNKS, step, 0)
        plsc.subcore_barrier()

        def wb(i, carry):
            r = rbase + i * CHUNK
            pltpu.sync_copy(acc_sh.at[pl.ds(r, CHUNK)], rows_v)
            pltpu.sync_copy(rows_v, out_hbm.at[c, pl.ds(r, CHUNK)])
            return carry

        lax.fori_loop(0, n_full, wb, 0)
        rt = rbase + n_full * CHUNK
        pltpu.sync_copy(acc_sh.at[pl.ds(rt, tail)], rows_v.at[pl.ds(0, tail)])
        pltpu.sync_copy(rows_v.at[pl.ds(0, tail)], out_hbm.at[c, pl.ds(rt, tail)])

    return pl.kernel(
        body,
        out_type=(jax.ShapeDtypeStruct((NC, N_PAD, D), jnp.float32),),
        mesh=mesh,
        scratch_types=(
            pltpu.VMEM((CHUNK,), jnp.int32),
            pltpu.VMEM((CHUNK,), jnp.int32),
            pltpu.VMEM((CHUNK, D), jnp.float32),
            pltpu.VMEM_SHARED((N_PAD, D), jnp.float32),
            pltpu.SemaphoreType.DMA,
        ),
    )


_deg_count = _make_deg_count()
_seg_sum = _make_seg_sum()


RB = 1000


def _dual_mm_body(x_ref, wl_ref, wr_ref, a_ref, b_ref):
    xb = x_ref[...]
    a_ref[...] = lax.dot_general(xb, wl_ref[...], _DN,
                                 preferred_element_type=jnp.float32)
    b_ref[...] = lax.dot_general(xb, wr_ref[...], _DN,
                                 preferred_element_type=jnp.float32)


def _dual_matmul(xx, Wl, Wr):
    n = xx.shape[0]
    return pl.pallas_call(
        _dual_mm_body,
        grid=(n // RB,),
        in_specs=[pl.BlockSpec((RB, D), lambda i: (i, 0)),
                  pl.BlockSpec((D, D), lambda i: (0, 0)),
                  pl.BlockSpec((D, D), lambda i: (0, 0))],
        out_specs=[pl.BlockSpec((RB, D), lambda i: (i, 0)),
                   pl.BlockSpec((RB, D), lambda i: (i, 0))],
        out_shape=[jax.ShapeDtypeStruct((n, D), jnp.float32)] * 2,
    )(xx, Wl, Wr)


def _mid_body(p_ref, dt_ref, xr_ref, b1_ref, wl_ref, wr_ref,
              ht_ref, hr_ref, dg_ref):
    pp = p_ref[...]
    ssum = pp[0] + pp[1]
    deg = jnp.sum(dt_ref[...], axis=1, keepdims=True)
    degc = jnp.maximum(deg, 1.0)
    aggr = ssum / degc
    h = jnp.maximum(aggr + b1_ref[...] + xr_ref[...], 0.0)
    ht_ref[...] = lax.dot_general(h, wl_ref[...], _DN,
                                  preferred_element_type=jnp.float32)
    hr_ref[...] = lax.dot_general(h, wr_ref[...], _DN,
                                  preferred_element_type=jnp.float32)
    dg_ref[...] = jnp.broadcast_to(degc, (RB, 8))


def _mid(p, dt, xr, b1r, W2l, W2r):
    return pl.pallas_call(
        _mid_body,
        grid=(N // RB,),
        in_specs=[pl.BlockSpec((NC, RB, D), lambda i: (0, i, 0)),
                  pl.BlockSpec((RB, NW), lambda i: (i, 0)),
                  pl.BlockSpec((RB, D), lambda i: (i, 0)),
                  pl.BlockSpec((1, D), lambda i: (0, 0)),
                  pl.BlockSpec((D, D), lambda i: (0, 0)),
                  pl.BlockSpec((D, D), lambda i: (0, 0))],
        out_specs=[pl.BlockSpec((RB, D), lambda i: (i, 0)),
                   pl.BlockSpec((RB, D), lambda i: (i, 0)),
                   pl.BlockSpec((RB, 8), lambda i: (i, 0))],
        out_shape=[jax.ShapeDtypeStruct((N, D), jnp.float32),
                   jax.ShapeDtypeStruct((N, D), jnp.float32),
                   jax.ShapeDtypeStruct((N, 8), jnp.float32)],
    )(p, dt, xr, b1r, W2l, W2r)


def _fin_body(p_ref, dg_ref, hr_ref, b2_ref, o_ref):
    pp = p_ref[...]
    ssum = pp[0] + pp[1]
    degc = dg_ref[:, 0:1]
    o_ref[...] = ssum / degc + b2_ref[...] + hr_ref[...]


def _fin(p, dg, hr, b2r):
    return pl.pallas_call(
        _fin_body,
        grid=(N // RB,),
        in_specs=[pl.BlockSpec((NC, RB, D), lambda i: (0, i, 0)),
                  pl.BlockSpec((RB, 8), lambda i: (i, 0)),
                  pl.BlockSpec((RB, D), lambda i: (i, 0)),
                  pl.BlockSpec((1, D), lambda i: (0, 0))],
        out_specs=pl.BlockSpec((RB, D), lambda i: (i, 0)),
        out_shape=jax.ShapeDtypeStruct((N, D), jnp.float32),
    )(p, dg, hr, b2r)


def kernel(x, edge_index, W1l, b1, W1r, W2l, b2, W2r):
    src = edge_index[0].astype(jnp.int32)
    dst = edge_index[1].astype(jnp.int32)
    src = jnp.concatenate([src, jnp.zeros((E_PAD - E,), jnp.int32)])
    dst = jnp.concatenate([dst, jnp.full((E_PAD - E,), N, jnp.int32)])
    b1r = b1.reshape(1, D)
    b2r = b2.reshape(1, D)

    xt, xr = _dual_matmul(x, W1l, W1r)
    (p1,) = _seg_sum(xt, src, dst)
    (deg_raw,) = _deg_count(dst)
    dt = deg_raw.reshape(NW, N_PAD).T
    ht, hr, dg = _mid(p1, dt, xr, b1r, W2l, W2r)
    (p2,) = _seg_sum(ht, src, dst)
    return _fin(p2, dg, hr, b2r)
